# baseline (speedup 1.0000x reference)
import functools

import jax
import jax.numpy as jnp
from jax import lax
from jax.experimental import pallas as pl
from jax.experimental.pallas import tpu as pltpu
from jax.experimental.pallas import tpu_sc as plsc

_NUM_CORES = 2
_NUM_SUBCORES = 16
_LANES = 16

_B = 4
_S = 8192
_D = 1024
_DB = 480
_DC = 32
_OUT_COLS = 2 * _DB + 1
_OUT_PAD = 2 * _DB + _LANES


def _body(table_hbm, ids_hbm, out_hbm, ids_v, idx_v, rows_v, out_v, red_v,
          sem):
    wid = lax.axis_index("s") * _NUM_CORES + lax.axis_index("c")

    @pl.when(wid == 0)
    def _():
        pltpu.sync_copy(ids_hbm, ids_v)
        ids = ids_v[...]
        lane = lax.iota(jnp.int32, _LANES)
        batch = lax.rem(lane, _B)
        flat = ids + batch * _S
        idx_v[...] = flat

        pltpu.async_copy(table_hbm.at[idx_v], rows_v, sem).wait()

        for j in range(2 * _DB // _LANES):
            src = 0 if j < _DB // _LANES else _B
            col = pl.ds(j * _LANES, _LANES)
            for b in range(_B):
                out_v[b, col] = rows_v[src + b, col]

        for b in range(_B):
            a0 = rows_v[b, pl.ds(2 * _DB, _LANES)]
            a1 = rows_v[b, pl.ds(2 * _DB + _LANES, _LANES)]
            e0 = rows_v[_B + b, pl.ds(2 * _DB + _DC, _LANES)]
            e1 = rows_v[_B + b, pl.ds(2 * _DB + _DC + _LANES, _LANES)]
            p = a0 * e0 + a1 * e1
            s = p[0]
            for i in range(1, _LANES):
                s = s + p[i]
            out_v[b, pl.ds(2 * _DB, _LANES)] = jnp.full((_LANES,), s,
                                                        jnp.float32)

        pltpu.sync_copy(out_v, out_hbm)


@jax.jit
def _run(table, ids):
    mesh = plsc.VectorSubcoreMesh(
        core_axis_name="c", subcore_axis_name="s",
        num_cores=_NUM_CORES, num_subcores=_NUM_SUBCORES)
    return pl.kernel(
        _body,
        out_type=jax.ShapeDtypeStruct((_B, _OUT_PAD), jnp.float32),
        mesh=mesh,
        scratch_types=[
            pltpu.VMEM((_LANES,), jnp.int32),
            pltpu.VMEM((_LANES,), jnp.int32),
            pltpu.VMEM((_LANES, _D), jnp.float32),
            pltpu.VMEM((_B, _OUT_PAD), jnp.float32),
            pltpu.VMEM((_LANES,), jnp.float32),
            pltpu.SemaphoreType.DMA,
        ],
    )(table, ids)


def kernel(encoded_input, start_ids, end_ids):
    table = encoded_input.reshape(_B * _S, _D)
    ids = jnp.zeros((_LANES,), jnp.int32)
    ids = ids.at[0:_B].set(start_ids.astype(jnp.int32))
    ids = ids.at[_B:2 * _B].set(end_ids.astype(jnp.int32))
    return _run(table, ids)[:, :_OUT_COLS]
